# baseline (speedup 1.0000x reference)
import functools

import jax
import jax.numpy as jnp
from jax import lax
from jax.experimental import pallas as pl
from jax.experimental.pallas import tpu as pltpu
from jax.experimental.pallas import tpu_sc as plsc

S, IN_F, OUT_F, E = 2048, 768, 768, 8
EPAD = 128
T = 256
NB = 16
SPAD = NB * T
RBLK = 128

NC, NS = 2, 16
NW = NC * NS
ROWS_W = S // NW


def _plan_body(x_ref, rw_ref, w_ref,
               xw_ref, wr_ref, dest_ref, be_ref, nb_ref, rank_ref):
    x = x_ref[...]
    logits = lax.dot_general(x, rw_ref[...], (((1,), (1,)), ((), ())),
                             preferred_element_type=jnp.float32)
    col = lax.broadcasted_iota(jnp.int32, (S, EPAD), 1)
    lg = jnp.where(col < E, logits, jnp.float32(-1e30))
    m = jnp.max(lg, axis=1, keepdims=True)
    denom = jnp.sum(jnp.exp(lg - m), axis=1, keepdims=True)
    top1w = 1.0 / denom
    idx = jnp.min(jnp.where(lg >= m, col, EPAD), axis=1, keepdims=True)
    oh = (col == idx).astype(jnp.float32)

    xw_ref[...] = x * top1w
    wr_ref[...] = jnp.clip(jnp.round(w_ref[...] * 2.0), -1.0, 1.0)

    ri = lax.broadcasted_iota(jnp.int32, (RBLK, RBLK), 0)
    ci = lax.broadcasted_iota(jnp.int32, (RBLK, RBLK), 1)
    ltri = (ri > ci).astype(jnp.float32)
    running = jnp.zeros((1, EPAD), jnp.float32)
    for b in range(S // RBLK):
        blk = oh[b * RBLK:(b + 1) * RBLK, :]
        r = lax.dot_general(ltri, blk, (((1,), (0,)), ((), ())),
                            preferred_element_type=jnp.float32) + running
        rank_ref[b * RBLK:(b + 1) * RBLK, :] = (
            jnp.sum(r * blk, axis=1, keepdims=True))
        running = running + jnp.sum(blk, axis=0, keepdims=True)
    counts = running

    tf = jnp.float32(T)
    pc = jnp.floor((counts + (tf - 1.0)) / tf) * tf
    ui = lax.broadcasted_iota(jnp.int32, (EPAD, EPAD), 0)
    uj = lax.broadcasted_iota(jnp.int32, (EPAD, EPAD), 1)
    utri = (ui < uj).astype(jnp.float32)
    pad_off = lax.dot_general(pc, utri, (((1,), (0,)), ((), ())),
                              preferred_element_type=jnp.float32)

    po_tok = jnp.sum(oh * pad_off, axis=1, keepdims=True)
    dest_ref[...] = (po_tok + rank_ref[...]).astype(jnp.int32)

    nbrow = lax.broadcasted_iota(jnp.int32, (NB, EPAD), 0).astype(jnp.float32)
    lane = lax.broadcasted_iota(jnp.int32, (NB, EPAD), 1)
    pbase = pad_off / tf
    pcap = pc / tf
    mfit = (nbrow >= pbase) & (nbrow < pbase + pcap)
    be_ref[...] = jnp.sum(
        jnp.where(mfit, lane, 0), axis=1, keepdims=True).astype(jnp.int32)
    nb_ref[...] = jnp.sum(pcap, axis=1, keepdims=True).astype(jnp.int32)


def _plan_call(x2, rw_pad, weight, interpret=False):
    return pl.pallas_call(
        _plan_body,
        out_shape=(
            jax.ShapeDtypeStruct((S, IN_F), jnp.float32),
            jax.ShapeDtypeStruct((OUT_F, IN_F), jnp.float32),
            jax.ShapeDtypeStruct((S, 1), jnp.int32),
            jax.ShapeDtypeStruct((NB, 1), jnp.int32),
            jax.ShapeDtypeStruct((1, 1), jnp.int32),
        ),
        scratch_shapes=[pltpu.VMEM((S, 1), jnp.float32)],
        interpret=interpret,
    )(x2, rw_pad, weight)


def _mm_body(be_ref, nb_ref, xs_ref, wr_ref, scale_ref, masks_ref, o_ref):
    @pl.when(pl.program_id(0) < nb_ref[0])
    def _():
        wm = wr_ref[...] * masks_ref[0]
        y = lax.dot_general(xs_ref[...], wm, (((1,), (1,)), ((), ())),
                            preferred_element_type=jnp.float32)
        o_ref[...] = y * scale_ref[...]


def _mm_call(be, nb, xs, wr, scale_row, masks, interpret=False):
    grid_spec = pltpu.PrefetchScalarGridSpec(
        num_scalar_prefetch=2,
        grid=(NB,),
        in_specs=[
            pl.BlockSpec((T, IN_F), lambda b, be_r, nb_r: (b, 0)),
            pl.BlockSpec((OUT_F, IN_F), lambda b, be_r, nb_r: (0, 0)),
            pl.BlockSpec((1, OUT_F), lambda b, be_r, nb_r: (0, 0)),
            pl.BlockSpec((1, OUT_F, IN_F),
                         lambda b, be_r, nb_r: (be_r[b], 0, 0)),
        ],
        out_specs=pl.BlockSpec((T, OUT_F), lambda b, be_r, nb_r: (b, 0)),
    )
    return pl.pallas_call(
        _mm_body,
        grid_spec=grid_spec,
        out_shape=jax.ShapeDtypeStruct((SPAD, OUT_F), jnp.float32),
        interpret=interpret,
    )(be, nb, xs, wr, scale_row, masks)


def _sc_mesh():
    return plsc.VectorSubcoreMesh(core_axis_name="c", subcore_axis_name="s")


def _scatter_call(xw, dest):
    @functools.partial(
        pl.kernel,
        mesh=_sc_mesh(),
        out_type=jax.ShapeDtypeStruct((SPAD, IN_F), jnp.float32),
        scratch_types=[
            pltpu.VMEM((ROWS_W,), jnp.int32),
            pltpu.VMEM((ROWS_W, IN_F), jnp.float32),
            pltpu.SemaphoreType.DMA,
        ],
    )
    def k(xw_hbm, dest_hbm, xs_hbm, idx_v, rows_v, sem):
        wid = lax.axis_index("s") * NC + lax.axis_index("c")
        base = wid * ROWS_W
        pltpu.sync_copy(dest_hbm.at[pl.ds(base, ROWS_W)], idx_v)
        pltpu.sync_copy(xw_hbm.at[pl.ds(base, ROWS_W)], rows_v)
        pltpu.async_copy(rows_v, xs_hbm.at[idx_v], sem).wait()

    return k(xw, dest)


def _gather_call(ys, dest):
    @functools.partial(
        pl.kernel,
        mesh=_sc_mesh(),
        out_type=jax.ShapeDtypeStruct((S, OUT_F), jnp.float32),
        scratch_types=[
            pltpu.VMEM((ROWS_W,), jnp.int32),
            pltpu.VMEM((ROWS_W, OUT_F), jnp.float32),
            pltpu.SemaphoreType.DMA,
        ],
    )
    def k(ys_hbm, dest_hbm, out_hbm, idx_v, rows_v, sem):
        wid = lax.axis_index("s") * NC + lax.axis_index("c")
        base = wid * ROWS_W
        pltpu.sync_copy(dest_hbm.at[pl.ds(base, ROWS_W)], idx_v)
        pltpu.async_copy(ys_hbm.at[idx_v], rows_v, sem).wait()
        pltpu.sync_copy(rows_v, out_hbm.at[pl.ds(base, ROWS_W)])

    return k(ys, dest)


@jax.jit
def kernel(x, weight, scale, threshold, expert_masks, router_w):
    del threshold
    x2 = x.reshape(S, IN_F)
    rw_pad = jnp.zeros((EPAD, IN_F), jnp.float32).at[:E].set(router_w)
    scale_row = scale.reshape(1, OUT_F)

    xw, wr, dest, be, nb = _plan_call(x2, rw_pad, weight)
    dest1 = dest.reshape(S)
    xs = _scatter_call(xw, dest1)
    ys = _mm_call(be.reshape(NB), nb.reshape(1), xs, wr, scale_row,
                  expert_masks)
    out = _gather_call(ys, dest1)
    return out.reshape(1, S, OUT_F)


# speedup vs baseline: 2.0259x; 2.0259x over previous
import jax
import jax.numpy as jnp
from jax import lax
from jax.experimental import pallas as pl
from jax.experimental.pallas import tpu as pltpu

S, IN_F, OUT_F, E = 2048, 768, 768, 8


def _fused_body(x_ref, rw_ref, w_ref, scale_ref, masks_ref, o_ref,
                xw_s, idx_s, wr_s):
    e = pl.program_id(0)

    @pl.when(e == 0)
    def _():
        x = x_ref[...]
        logits = lax.dot_general(x, rw_ref[...], (((1,), (1,)), ((), ())),
                                 preferred_element_type=jnp.float32)
        col = lax.broadcasted_iota(jnp.int32, (S, E), 1)
        m = jnp.max(logits, axis=1, keepdims=True)
        top1w = 1.0 / jnp.sum(jnp.exp(logits - m), axis=1, keepdims=True)
        idx_s[...] = jnp.min(jnp.where(logits >= m, col, E), axis=1,
                             keepdims=True)
        xw_s[...] = x * top1w
        wr_s[...] = jnp.clip(jnp.round(w_ref[...] * 2.0), -1.0, 1.0)
        o_ref[...] = jnp.zeros((S, OUT_F), jnp.float32)

    wm = wr_s[...] * masks_ref[0]
    ye = lax.dot_general(xw_s[...], wm, (((1,), (1,)), ((), ())),
                         preferred_element_type=jnp.float32)
    sel = (idx_s[...] == e).astype(jnp.float32)
    o_ref[...] += sel * (ye * scale_ref[...])


def _fused_call(x2, router_w, weight, scale_row, masks, interpret=False):
    return pl.pallas_call(
        _fused_body,
        grid=(E,),
        in_specs=[
            pl.BlockSpec((S, IN_F), lambda e: (0, 0)),
            pl.BlockSpec((E, IN_F), lambda e: (0, 0)),
            pl.BlockSpec((OUT_F, IN_F), lambda e: (0, 0)),
            pl.BlockSpec((1, OUT_F), lambda e: (0, 0)),
            pl.BlockSpec((1, OUT_F, IN_F), lambda e: (e, 0, 0)),
        ],
        out_specs=pl.BlockSpec((S, OUT_F), lambda e: (0, 0)),
        out_shape=jax.ShapeDtypeStruct((S, OUT_F), jnp.float32),
        scratch_shapes=[
            pltpu.VMEM((S, IN_F), jnp.float32),
            pltpu.VMEM((S, 1), jnp.int32),
            pltpu.VMEM((OUT_F, IN_F), jnp.float32),
        ],
        interpret=interpret,
    )(x2, router_w, weight, scale_row, masks)


@jax.jit
def kernel(x, weight, scale, threshold, expert_masks, router_w):
    del threshold
    out = _fused_call(x.reshape(S, IN_F), router_w, weight,
                      scale.reshape(1, OUT_F), expert_masks)
    return out.reshape(1, S, OUT_F)


# speedup vs baseline: 2.0266x; 1.0003x over previous
import jax
import jax.numpy as jnp
from jax import lax
from jax.experimental import pallas as pl
from jax.experimental.pallas import tpu as pltpu

S, IN_F, OUT_F, E = 2048, 768, 768, 8


def _fused_body(x_ref, rw_ref, w_ref, scale_ref, masks_ref, o_ref,
                xw_s, idx_s, wr_s):
    e = pl.program_id(0)

    @pl.when(e == 0)
    def _():
        x = x_ref[...]
        logits = lax.dot_general(x, rw_ref[...], (((1,), (1,)), ((), ())),
                                 preferred_element_type=jnp.float32)
        col = lax.broadcasted_iota(jnp.int32, (S, E), 1)
        m = jnp.max(logits, axis=1, keepdims=True)
        top1w = 1.0 / jnp.sum(jnp.exp(logits - m), axis=1, keepdims=True)
        idx_s[...] = jnp.min(jnp.where(logits >= m, col, E), axis=1,
                             keepdims=True)
        xw_s[...] = (x * top1w).astype(jnp.bfloat16)
        wr_s[...] = jnp.clip(jnp.round(w_ref[...] * 2.0),
                             -1.0, 1.0).astype(jnp.bfloat16)
        o_ref[...] = jnp.zeros((S, OUT_F), jnp.float32)

    wm = wr_s[...] * masks_ref[0].astype(jnp.bfloat16)
    ye = lax.dot_general(xw_s[...], wm, (((1,), (1,)), ((), ())),
                         preferred_element_type=jnp.float32)
    sel = (idx_s[...] == e).astype(jnp.float32)
    o_ref[...] += sel * (ye * scale_ref[...])


def _fused_call(x2, router_w, weight, scale_row, masks, interpret=False):
    return pl.pallas_call(
        _fused_body,
        grid=(E,),
        in_specs=[
            pl.BlockSpec((S, IN_F), lambda e: (0, 0)),
            pl.BlockSpec((E, IN_F), lambda e: (0, 0)),
            pl.BlockSpec((OUT_F, IN_F), lambda e: (0, 0)),
            pl.BlockSpec((1, OUT_F), lambda e: (0, 0)),
            pl.BlockSpec((1, OUT_F, IN_F), lambda e: (e, 0, 0)),
        ],
        out_specs=pl.BlockSpec((S, OUT_F), lambda e: (0, 0)),
        out_shape=jax.ShapeDtypeStruct((S, OUT_F), jnp.float32),
        scratch_shapes=[
            pltpu.VMEM((S, IN_F), jnp.bfloat16),
            pltpu.VMEM((S, 1), jnp.int32),
            pltpu.VMEM((OUT_F, IN_F), jnp.bfloat16),
        ],
        interpret=interpret,
    )(x2, router_w, weight, scale_row, masks)


@jax.jit
def kernel(x, weight, scale, threshold, expert_masks, router_w):
    del threshold
    out = _fused_call(x.reshape(S, IN_F), router_w, weight,
                      scale.reshape(1, OUT_F), expert_masks)
    return out.reshape(1, S, OUT_F)
